# Initial kernel scaffold; baseline (speedup 1.0000x reference)
import jax
import jax.numpy as jnp
from jax.experimental import pallas as pl


def kernel(node_input, node_attr, node_deg, edge_src, edge_dst, edge_attr, edge_length_embedded, numb, n, W_in, W_mask, W_r0, W_r1, W_out):
    raise NotImplementedError("write your pallas kernel here")


# speedup vs baseline: 1.5866x; 1.5866x over previous
import math

import jax
import jax.numpy as jnp
from jax import lax
from jax.experimental import pallas as pl
from jax.experimental.pallas import tpu as pltpu
from jax.experimental.pallas import tpu_sc as plsc

NC = 2
NS = 16
NT = NC * NS
LANES = 16
CH = 128


def _node_feat_body(x_ref, attr_ref, deg_ref, w_ref, nf_ref):
    x = x_ref[...]
    scale = attr_ref[...] * lax.rsqrt(deg_ref[...]) * (1.0 / math.sqrt(x.shape[1]))
    nf_ref[...] = jnp.dot(x, w_ref[...], preferred_element_type=jnp.float32) * scale


def _edge_mlp_body(elb_ref, ea_ref, wr0_ref, wr1_ref, out_ref):
    nb = elb_ref.shape[1]
    h = jnp.dot(elb_ref[...], wr0_ref[...], preferred_element_type=jnp.float32)
    h = h * (1.0 / math.sqrt(nb))
    h = h * jax.nn.sigmoid(h)
    hdim = wr0_ref.shape[1]
    ew = jnp.dot(h, wr1_ref[...], preferred_element_type=jnp.float32)
    out_ref[...] = ew * (ea_ref[...] * (1.0 / math.sqrt(hdim)))


def _final_body(x_ref, attr_ref, deg_ref, parts_ref, wm_ref, wo_ref, out_ref):
    c_s, c_x = math.sin(math.pi / 8), math.cos(math.pi / 8)
    d = x_ref.shape[1]
    attr = attr_ref[...] * (1.0 / math.sqrt(d))
    nm = jnp.dot(x_ref[...], wm_ref[...], preferred_element_type=jnp.float32) * attr
    nf2 = (parts_ref[0] + parts_ref[1]) * lax.rsqrt(deg_ref[...])
    no = jnp.dot(nf2, wo_ref[...], preferred_element_type=jnp.float32) * attr
    out_ref[...] = c_s * nm + c_x * no


def _idx_group(k_chunks):
    return next(z for z in range(min(8, k_chunks), 0, -1) if k_chunks % z == 0)


def _make_sc_body(n_pad, d, ept, k_chunks):
    rps = n_pad // NS
    n_colv = d // LANES

    def body(nf_hbm, ewp_hbm, src_hbm, dst_hbm, out_hbm,
             acc, ew_v, g_v, src_v, dst_v, sem):
        c = lax.axis_index("c")
        s = lax.axis_index("s")
        tile = c * NS + s

        def zrow(i, carry):
            for cc in range(n_colv):
                g_v[i, pl.ds(cc * LANES, LANES)] = jnp.zeros((LANES,), jnp.float32)
            return carry
        lax.fori_loop(0, CH, zrow, 0)
        row0 = s * rps
        for kk in range(rps // CH):
            pltpu.sync_copy(g_v, acc.at[pl.ds(row0 + kk * CH, CH)])
        plsc.subcore_barrier()

        grp = _idx_group(k_chunks)

        def group(g, carry):
            pltpu.sync_copy(src_hbm.at[tile, pl.ds(g * grp, grp)], src_v)
            pltpu.sync_copy(dst_hbm.at[tile, pl.ds(g * grp, grp)], dst_v)

            def chunk(jj, carry2):
                e0 = tile * ept + (g * grp + jj) * CH
                pltpu.sync_copy(ewp_hbm.at[pl.ds(e0, CH)], ew_v)
                pltpu.async_copy(nf_hbm.at[src_v.at[jj]], g_v, sem).wait()

                def mrow(r, carry3):
                    for cc in range(n_colv):
                        sl = pl.ds(cc * LANES, LANES)
                        g_v[r, sl] = g_v[r, sl] * ew_v[r, sl]
                    return carry3
                lax.fori_loop(0, CH, mrow, 0)
                pltpu.sync_copy(g_v, acc.at[dst_v.at[jj]], add=True)
                return carry2
            lax.fori_loop(0, grp, chunk, 0)
            return carry
        lax.fori_loop(0, k_chunks // grp, group, 0)

        plsc.subcore_barrier()
        pltpu.sync_copy(acc.at[pl.ds(row0, rps)], out_hbm.at[c, pl.ds(row0, rps)])

    return body


def kernel(node_input, node_attr, node_deg, edge_src, edge_dst, edge_attr,
           edge_length_embedded, numb, n, W_in, W_mask, W_r0, W_r1, W_out):
    n_nodes, d = node_input.shape
    e = edge_src.shape[0]
    nb = edge_length_embedded.shape[1]

    bn = 1000
    nf = pl.pallas_call(
        _node_feat_body,
        grid=(n_nodes // bn,),
        in_specs=[
            pl.BlockSpec((bn, d), lambda i: (i, 0)),
            pl.BlockSpec((bn, 1), lambda i: (i, 0)),
            pl.BlockSpec((bn, 1), lambda i: (i, 0)),
            pl.BlockSpec((d, d), lambda i: (0, 0)),
        ],
        out_specs=pl.BlockSpec((bn, d), lambda i: (i, 0)),
        out_shape=jax.ShapeDtypeStruct((n_nodes, d), jnp.float32),
    )(node_input, node_attr, node_deg, W_in)

    ept = ((e + NT * 256 - 1) // (NT * 256)) * 256
    k_chunks = ept // CH
    e_pad = NT * ept
    pad = e_pad - e
    elb_p = jnp.concatenate(
        [edge_length_embedded, jnp.zeros((pad, nb), jnp.float32)])
    ea_p = jnp.concatenate([edge_attr, jnp.zeros((pad, 1), jnp.float32)])
    src_r = jnp.concatenate(
        [edge_src, jnp.zeros((pad,), jnp.int32)]).reshape(NT, k_chunks, CH)
    dst_r = jnp.concatenate(
        [edge_dst, jnp.zeros((pad,), jnp.int32)]).reshape(NT, k_chunks, CH)

    be = 8192
    hdim = W_r0.shape[1]
    ewp = pl.pallas_call(
        _edge_mlp_body,
        grid=(e_pad // be,),
        in_specs=[
            pl.BlockSpec((be, nb), lambda i: (i, 0)),
            pl.BlockSpec((be, 1), lambda i: (i, 0)),
            pl.BlockSpec((nb, hdim), lambda i: (0, 0)),
            pl.BlockSpec((hdim, d), lambda i: (0, 0)),
        ],
        out_specs=pl.BlockSpec((be, d), lambda i: (i, 0)),
        out_shape=jax.ShapeDtypeStruct((e_pad, d), jnp.float32),
    )(elb_p, ea_p, W_r0, W_r1)

    n_pad = ((n_nodes + NS * CH - 1) // (NS * CH)) * (NS * CH)
    mesh = plsc.VectorSubcoreMesh(core_axis_name="c", subcore_axis_name="s")
    parts = pl.kernel(
        _make_sc_body(n_pad, d, ept, k_chunks),
        out_type=jax.ShapeDtypeStruct((NC, n_pad, d), jnp.float32),
        mesh=mesh,
        scratch_types=[
            pltpu.VMEM_SHARED((n_pad, d), jnp.float32),
            pltpu.VMEM((CH, d), jnp.float32),
            pltpu.VMEM((CH, d), jnp.float32),
            pltpu.VMEM((_idx_group(k_chunks), CH), jnp.int32),
            pltpu.VMEM((_idx_group(k_chunks), CH), jnp.int32),
            pltpu.SemaphoreType.DMA,
        ],
    )(nf, ewp, src_r, dst_r)

    out = pl.pallas_call(
        _final_body,
        grid=(n_nodes // bn,),
        in_specs=[
            pl.BlockSpec((bn, d), lambda i: (i, 0)),
            pl.BlockSpec((bn, 1), lambda i: (i, 0)),
            pl.BlockSpec((bn, 1), lambda i: (i, 0)),
            pl.BlockSpec((NC, bn, d), lambda i: (0, i, 0)),
            pl.BlockSpec((d, d), lambda i: (0, 0)),
            pl.BlockSpec((d, d), lambda i: (0, 0)),
        ],
        out_specs=pl.BlockSpec((bn, d), lambda i: (i, 0)),
        out_shape=jax.ShapeDtypeStruct((n_nodes, d), jnp.float32),
    )(node_input, node_attr, node_deg, parts, W_mask, W_out)
    return out


# speedup vs baseline: 1.8566x; 1.1702x over previous
import math

import jax
import jax.numpy as jnp
from jax import lax
from jax.experimental import pallas as pl
from jax.experimental.pallas import tpu as pltpu
from jax.experimental.pallas import tpu_sc as plsc

NC = 2
NS = 16
NT = NC * NS
LANES = 16
CH = 64
GRP = 8


def _node_feat_body(x_ref, attr_ref, deg_ref, w_ref, nf_ref):
    x = x_ref[...]
    scale = attr_ref[...] * lax.rsqrt(deg_ref[...]) * (1.0 / math.sqrt(x.shape[1]))
    nf_ref[...] = jnp.dot(x, w_ref[...], preferred_element_type=jnp.float32) * scale


def _edge_mlp_body(elb_ref, ea_ref, wr0_ref, wr1_ref, out_ref):
    nb = elb_ref.shape[1]
    h = jnp.dot(elb_ref[...], wr0_ref[...], preferred_element_type=jnp.float32)
    h = h * (1.0 / math.sqrt(nb))
    h = h * jax.nn.sigmoid(h)
    hdim = wr0_ref.shape[1]
    ew = jnp.dot(h, wr1_ref[...], preferred_element_type=jnp.float32)
    out_ref[...] = ew * (ea_ref[...] * (1.0 / math.sqrt(hdim)))


def _final_body(x_ref, attr_ref, deg_ref, parts_ref, wm_ref, wo_ref, out_ref):
    c_s, c_x = math.sin(math.pi / 8), math.cos(math.pi / 8)
    d = x_ref.shape[1]
    attr = attr_ref[...] * (1.0 / math.sqrt(d))
    nm = jnp.dot(x_ref[...], wm_ref[...], preferred_element_type=jnp.float32) * attr
    nf2 = (parts_ref[0] + parts_ref[1]) * lax.rsqrt(deg_ref[...])
    no = jnp.dot(nf2, wo_ref[...], preferred_element_type=jnp.float32) * attr
    out_ref[...] = c_s * nm + c_x * no


def _make_sc_body(n_pad, d, ept, k_chunks):
    rps = n_pad // NS
    n_colv = d // LANES

    def body(nf_hbm, ewp_hbm, src_hbm, dst_hbm, out_hbm,
             acc, ew_v, g_v, src_v, dst_v, sem_ew, sem_g, sem_sc):
        c = lax.axis_index("c")
        s = lax.axis_index("s")
        tile = c * NS + s

        def stage_idx(grp_i):
            gb = (grp_i // GRP) % 2
            g0 = pl.multiple_of(grp_i, GRP)
            pltpu.sync_copy(src_hbm.at[tile, pl.ds(g0, GRP)], src_v.at[gb])
            pltpu.sync_copy(dst_hbm.at[tile, pl.ds(g0, GRP)], dst_v.at[gb])

        def issue_ew(t):
            b = t % 2
            e0 = tile * ept + t * CH
            pltpu.async_copy(ewp_hbm.at[pl.ds(e0, CH)], ew_v.at[b], sem_ew.at[b])

        def wait_ew(t):
            b = t % 2
            pltpu.make_async_copy(
                ewp_hbm.at[pl.ds(0, CH)], ew_v.at[b], sem_ew.at[b]).wait()

        def issue_gather(t):
            b = t % 3
            idx = src_v.at[(t // GRP) % 2, t % GRP]
            pltpu.async_copy(nf_hbm.at[idx], g_v.at[b], sem_g.at[b])

        def wait_gather(t):
            b = t % 3
            pltpu.make_async_copy(
                nf_hbm.at[src_v.at[0, 0]], g_v.at[b], sem_g.at[b]).wait()

        def issue_scatter(t):
            b = t % 3
            idx = dst_v.at[(t // GRP) % 2, t % GRP]
            pltpu.async_copy(g_v.at[b], acc.at[idx], sem_sc.at[b], add=True)

        def wait_scatter(b):
            pltpu.make_async_copy(
                g_v.at[b], acc.at[dst_v.at[0, 0]], sem_sc.at[b]).wait()

        def zrow(i, carry):
            for cc in range(n_colv):
                g_v[0, i, pl.ds(cc * LANES, LANES)] = jnp.zeros(
                    (LANES,), jnp.float32)
            return carry
        lax.fori_loop(0, CH, zrow, 0)
        row0 = s * rps
        for kk in range(rps // CH):
            pltpu.sync_copy(g_v.at[0], acc.at[pl.ds(row0 + kk * CH, CH)])
        plsc.subcore_barrier()

        stage_idx(0)
        issue_ew(0)
        issue_gather(0)

        def step(j, carry):
            t = j + 1

            @pl.when(t < k_chunks)
            def _():
                @pl.when(lax.rem(t, GRP) == 0)
                def _():
                    stage_idx(t)
                issue_ew(t)

                @pl.when(t >= 3)
                def _():
                    wait_scatter(t % 3)
                issue_gather(t)

            wait_ew(j)
            wait_gather(j)
            b3 = j % 3
            b2 = j % 2

            @plsc.parallel_loop(0, CH, 1, unroll=4)
            def mrow(r):
                for cc in range(n_colv):
                    sl = pl.ds(cc * LANES, LANES)
                    g_v[b3, r, sl] = g_v[b3, r, sl] * ew_v[b2, r, sl]

            issue_scatter(j)
            return carry
        lax.fori_loop(0, k_chunks, step, 0)
        for b in range(3):
            wait_scatter(b)

        plsc.subcore_barrier()
        pltpu.sync_copy(acc.at[pl.ds(row0, rps)], out_hbm.at[c, pl.ds(row0, rps)])

    return body


def kernel(node_input, node_attr, node_deg, edge_src, edge_dst, edge_attr,
           edge_length_embedded, numb, n, W_in, W_mask, W_r0, W_r1, W_out):
    n_nodes, d = node_input.shape
    e = edge_src.shape[0]
    nb = edge_length_embedded.shape[1]

    bn = 1000
    nf = pl.pallas_call(
        _node_feat_body,
        grid=(n_nodes // bn,),
        in_specs=[
            pl.BlockSpec((bn, d), lambda i: (i, 0)),
            pl.BlockSpec((bn, 1), lambda i: (i, 0)),
            pl.BlockSpec((bn, 1), lambda i: (i, 0)),
            pl.BlockSpec((d, d), lambda i: (0, 0)),
        ],
        out_specs=pl.BlockSpec((bn, d), lambda i: (i, 0)),
        out_shape=jax.ShapeDtypeStruct((n_nodes, d), jnp.float32),
    )(node_input, node_attr, node_deg, W_in)

    align = max(256, CH * GRP)
    ept = ((e + NT * align - 1) // (NT * align)) * align
    k_chunks = ept // CH
    e_pad = NT * ept
    pad = e_pad - e
    elb_p = jnp.concatenate(
        [edge_length_embedded, jnp.zeros((pad, nb), jnp.float32)])
    ea_p = jnp.concatenate([edge_attr, jnp.zeros((pad, 1), jnp.float32)])
    src_r = jnp.concatenate(
        [edge_src, jnp.zeros((pad,), jnp.int32)]).reshape(NT, k_chunks, CH)
    dst_r = jnp.concatenate(
        [edge_dst, jnp.zeros((pad,), jnp.int32)]).reshape(NT, k_chunks, CH)

    be = 8192
    hdim = W_r0.shape[1]
    ewp = pl.pallas_call(
        _edge_mlp_body,
        grid=(e_pad // be,),
        in_specs=[
            pl.BlockSpec((be, nb), lambda i: (i, 0)),
            pl.BlockSpec((be, 1), lambda i: (i, 0)),
            pl.BlockSpec((nb, hdim), lambda i: (0, 0)),
            pl.BlockSpec((hdim, d), lambda i: (0, 0)),
        ],
        out_specs=pl.BlockSpec((be, d), lambda i: (i, 0)),
        out_shape=jax.ShapeDtypeStruct((e_pad, d), jnp.float32),
    )(elb_p, ea_p, W_r0, W_r1)

    n_pad = ((n_nodes + NS * CH - 1) // (NS * CH)) * (NS * CH)
    mesh = plsc.VectorSubcoreMesh(core_axis_name="c", subcore_axis_name="s")
    parts = pl.kernel(
        _make_sc_body(n_pad, d, ept, k_chunks),
        out_type=jax.ShapeDtypeStruct((NC, n_pad, d), jnp.float32),
        mesh=mesh,
        scratch_types=[
            pltpu.VMEM_SHARED((n_pad, d), jnp.float32),
            pltpu.VMEM((2, CH, d), jnp.float32),
            pltpu.VMEM((3, CH, d), jnp.float32),
            pltpu.VMEM((2, GRP, CH), jnp.int32),
            pltpu.VMEM((2, GRP, CH), jnp.int32),
            pltpu.SemaphoreType.DMA((2,)),
            pltpu.SemaphoreType.DMA((3,)),
            pltpu.SemaphoreType.DMA((3,)),
        ],
    )(nf, ewp, src_r, dst_r)

    out = pl.pallas_call(
        _final_body,
        grid=(n_nodes // bn,),
        in_specs=[
            pl.BlockSpec((bn, d), lambda i: (i, 0)),
            pl.BlockSpec((bn, 1), lambda i: (i, 0)),
            pl.BlockSpec((bn, 1), lambda i: (i, 0)),
            pl.BlockSpec((NC, bn, d), lambda i: (0, i, 0)),
            pl.BlockSpec((d, d), lambda i: (0, 0)),
            pl.BlockSpec((d, d), lambda i: (0, 0)),
        ],
        out_specs=pl.BlockSpec((bn, d), lambda i: (i, 0)),
        out_shape=jax.ShapeDtypeStruct((n_nodes, d), jnp.float32),
    )(node_input, node_attr, node_deg, parts, W_mask, W_out)
    return out


# speedup vs baseline: 1.9214x; 1.0349x over previous
import math

import jax
import jax.numpy as jnp
from jax import lax
from jax.experimental import pallas as pl
from jax.experimental.pallas import tpu as pltpu
from jax.experimental.pallas import tpu_sc as plsc

NC = 2
NS = 16
NT = NC * NS
LANES = 16
CH = 64
GRP = 8


def _pack_bf16_pair(x):
    h = x.shape[1] // 2
    lo = lax.bitcast_convert_type(
        x[:, :h].astype(jnp.bfloat16), jnp.uint16).astype(jnp.uint32)
    hi = lax.bitcast_convert_type(
        x[:, h:].astype(jnp.bfloat16), jnp.uint16).astype(jnp.uint32)
    return lax.bitcast_convert_type(lo | (hi << 16), jnp.int32)


def _node_feat_body(x_ref, attr_ref, deg_ref, w_ref, nf_ref):
    x = x_ref[...]
    scale = attr_ref[...] * lax.rsqrt(deg_ref[...]) * (1.0 / math.sqrt(x.shape[1]))
    nf = jnp.dot(x, w_ref[...], preferred_element_type=jnp.float32) * scale
    nf_ref[...] = nf


def _edge_mlp_body(elb_ref, ea_ref, wr0_ref, wr1_ref, out_ref):
    nb = elb_ref.shape[1]
    h = jnp.dot(elb_ref[...], wr0_ref[...], preferred_element_type=jnp.float32)
    h = h * (1.0 / math.sqrt(nb))
    h = h * jax.nn.sigmoid(h)
    hdim = wr0_ref.shape[1]
    ew = jnp.dot(h, wr1_ref[...], preferred_element_type=jnp.float32)
    out_ref[...] = _pack_bf16_pair(ew * (ea_ref[...] * (1.0 / math.sqrt(hdim))))


def _final_body(x_ref, attr_ref, deg_ref, parts_ref, wm_ref, wo_ref, out_ref):
    c_s, c_x = math.sin(math.pi / 8), math.cos(math.pi / 8)
    d = x_ref.shape[1]
    attr = attr_ref[...] * (1.0 / math.sqrt(d))
    nm = jnp.dot(x_ref[...], wm_ref[...], preferred_element_type=jnp.float32) * attr
    nf2 = (parts_ref[0] + parts_ref[1]) * lax.rsqrt(deg_ref[...])
    no = jnp.dot(nf2, wo_ref[...], preferred_element_type=jnp.float32) * attr
    out_ref[...] = c_s * nm + c_x * no


def _make_sc_body(n_pad, d, ept, k_chunks):
    rps = n_pad // NS
    half = d // 2

    def body(nf_hbm, ewp_hbm, src_hbm, dst_hbm, out_hbm,
             acc, ew_v, g_v, src_v, dst_v, sem_ew, sem_g, sem_sc):
        c = lax.axis_index("c")
        s = lax.axis_index("s")
        tile = c * NS + s

        def stage_idx(grp_i):
            gb = (grp_i // GRP) % 2
            g0 = pl.multiple_of(grp_i, GRP)
            pltpu.sync_copy(src_hbm.at[tile, pl.ds(g0, GRP)], src_v.at[gb])
            pltpu.sync_copy(dst_hbm.at[tile, pl.ds(g0, GRP)], dst_v.at[gb])

        def issue_ew(t):
            b = t % 2
            e0 = tile * ept + t * CH
            pltpu.async_copy(ewp_hbm.at[pl.ds(e0, CH)], ew_v.at[b], sem_ew.at[b])

        def wait_ew(t):
            b = t % 2
            pltpu.make_async_copy(
                ewp_hbm.at[pl.ds(0, CH)], ew_v.at[b], sem_ew.at[b]).wait()

        def issue_gather(t):
            b = t % 2
            idx = src_v.at[(t // GRP) % 2, t % GRP]
            pltpu.async_copy(nf_hbm.at[idx], g_v.at[b], sem_g.at[b])

        def wait_gather(t):
            b = t % 2
            pltpu.make_async_copy(
                nf_hbm.at[src_v.at[0, 0]], g_v.at[b], sem_g.at[b]).wait()

        def issue_scatter(t):
            b = t % 2
            idx = dst_v.at[(t // GRP) % 2, t % GRP]
            pltpu.async_copy(g_v.at[b], acc.at[idx], sem_sc.at[b], add=True)

        def wait_scatter(b):
            pltpu.make_async_copy(
                g_v.at[b], acc.at[dst_v.at[0, 0]], sem_sc.at[b]).wait()

        def zrow(r, carry):
            for cc in range(d // LANES):
                g_v[0, r, pl.ds(cc * LANES, LANES)] = jnp.zeros(
                    (LANES,), jnp.float32)
            return carry
        lax.fori_loop(0, CH, zrow, 0)
        row0 = s * rps
        for kk in range(rps // CH):
            pltpu.sync_copy(g_v.at[0], acc.at[pl.ds(row0 + kk * CH, CH)])
        rem = rps % CH
        if rem:
            pltpu.sync_copy(g_v.at[0, pl.ds(0, rem)],
                            acc.at[pl.ds(row0 + (rps // CH) * CH, rem)])
        plsc.subcore_barrier()

        stage_idx(0)
        issue_ew(0)
        issue_gather(0)
        mask = jnp.int32(-65536)

        def step(j, carry):
            t = j + 1

            @pl.when(t < k_chunks)
            def _():
                @pl.when(lax.rem(t, GRP) == 0)
                def _():
                    stage_idx(t)
                issue_ew(t)

                @pl.when(t >= 2)
                def _():
                    wait_scatter(t % 2)
                issue_gather(t)

            wait_ew(j)
            wait_gather(j)
            b2 = j % 2

            @plsc.parallel_loop(0, CH, 1, unroll=2)
            def mrow(r):
                for g16 in range(half // LANES):
                    lo_sl = pl.ds(g16 * LANES, LANES)
                    hi_sl = pl.ds(half + g16 * LANES, LANES)
                    ew = ew_v[b2, r, lo_sl]
                    elo = lax.bitcast_convert_type(ew << 16, jnp.float32)
                    ehi = lax.bitcast_convert_type(ew & mask, jnp.float32)
                    g_v[b2, r, lo_sl] = g_v[b2, r, lo_sl] * elo
                    g_v[b2, r, hi_sl] = g_v[b2, r, hi_sl] * ehi

            issue_scatter(j)
            return carry
        lax.fori_loop(0, k_chunks, step, 0)
        for b in range(2):
            wait_scatter(b)

        plsc.subcore_barrier()
        pltpu.sync_copy(acc.at[pl.ds(row0, rps)], out_hbm.at[c, pl.ds(row0, rps)])

    return body


def kernel(node_input, node_attr, node_deg, edge_src, edge_dst, edge_attr,
           edge_length_embedded, numb, n, W_in, W_mask, W_r0, W_r1, W_out):
    n_nodes, d = node_input.shape
    e = edge_src.shape[0]
    nb = edge_length_embedded.shape[1]

    bn = 1000
    nf = pl.pallas_call(
        _node_feat_body,
        grid=(n_nodes // bn,),
        in_specs=[
            pl.BlockSpec((bn, d), lambda i: (i, 0)),
            pl.BlockSpec((bn, 1), lambda i: (i, 0)),
            pl.BlockSpec((bn, 1), lambda i: (i, 0)),
            pl.BlockSpec((d, d), lambda i: (0, 0)),
        ],
        out_specs=pl.BlockSpec((bn, d), lambda i: (i, 0)),
        out_shape=jax.ShapeDtypeStruct((n_nodes, d), jnp.float32),
    )(node_input, node_attr, node_deg, W_in)

    align = max(256, CH * GRP)
    ept = ((e + NT * align - 1) // (NT * align)) * align
    k_chunks = ept // CH
    e_pad = NT * ept
    pad = e_pad - e
    elb_p = jnp.concatenate(
        [edge_length_embedded, jnp.zeros((pad, nb), jnp.float32)])
    ea_p = jnp.concatenate([edge_attr, jnp.zeros((pad, 1), jnp.float32)])
    src_r = jnp.concatenate(
        [edge_src, jnp.zeros((pad,), jnp.int32)]).reshape(NT, k_chunks, CH)
    dst_r = jnp.concatenate(
        [edge_dst, jnp.zeros((pad,), jnp.int32)]).reshape(NT, k_chunks, CH)

    be = 8192
    hdim = W_r0.shape[1]
    ewp = pl.pallas_call(
        _edge_mlp_body,
        grid=(e_pad // be,),
        in_specs=[
            pl.BlockSpec((be, nb), lambda i: (i, 0)),
            pl.BlockSpec((be, 1), lambda i: (i, 0)),
            pl.BlockSpec((nb, hdim), lambda i: (0, 0)),
            pl.BlockSpec((hdim, d), lambda i: (0, 0)),
        ],
        out_specs=pl.BlockSpec((be, d // 2), lambda i: (i, 0)),
        out_shape=jax.ShapeDtypeStruct((e_pad, d // 2), jnp.int32),
    )(elb_p, ea_p, W_r0, W_r1)

    n_pad = ((n_nodes + NS * 8 - 1) // (NS * 8)) * (NS * 8)
    mesh = plsc.VectorSubcoreMesh(core_axis_name="c", subcore_axis_name="s")
    parts = pl.kernel(
        _make_sc_body(n_pad, d, ept, k_chunks),
        out_type=jax.ShapeDtypeStruct((NC, n_pad, d), jnp.float32),
        mesh=mesh,
        scratch_types=[
            pltpu.VMEM_SHARED((n_pad, d), jnp.float32),
            pltpu.VMEM((2, CH, d // 2), jnp.int32),
            pltpu.VMEM((2, CH, d), jnp.float32),
            pltpu.VMEM((2, GRP, CH), jnp.int32),
            pltpu.VMEM((2, GRP, CH), jnp.int32),
            pltpu.SemaphoreType.DMA((2,)),
            pltpu.SemaphoreType.DMA((2,)),
            pltpu.SemaphoreType.DMA((2,)),
        ],
    )(nf, ewp, src_r, dst_r)

    out = pl.pallas_call(
        _final_body,
        grid=(n_nodes // bn,),
        in_specs=[
            pl.BlockSpec((bn, d), lambda i: (i, 0)),
            pl.BlockSpec((bn, 1), lambda i: (i, 0)),
            pl.BlockSpec((bn, 1), lambda i: (i, 0)),
            pl.BlockSpec((NC, bn, d), lambda i: (0, i, 0)),
            pl.BlockSpec((d, d), lambda i: (0, 0)),
            pl.BlockSpec((d, d), lambda i: (0, 0)),
        ],
        out_specs=pl.BlockSpec((bn, d), lambda i: (i, 0)),
        out_shape=jax.ShapeDtypeStruct((n_nodes, d), jnp.float32),
    )(node_input, node_attr, node_deg, parts, W_mask, W_out)
    return out


# speedup vs baseline: 2.1926x; 1.1412x over previous
import math

import jax
import jax.numpy as jnp
from jax import lax
from jax.experimental import pallas as pl
from jax.experimental.pallas import tpu as pltpu
from jax.experimental.pallas import tpu_sc as plsc

NC = 2
NS = 16
NT = NC * NS
LANES = 16
CH = 64
GRP = 8


def _pack_bf16_pair(x):
    h = x.shape[1] // 2
    lo = lax.bitcast_convert_type(
        x[:, :h].astype(jnp.bfloat16), jnp.uint16).astype(jnp.uint32)
    hi = lax.bitcast_convert_type(
        x[:, h:].astype(jnp.bfloat16), jnp.uint16).astype(jnp.uint32)
    return lax.bitcast_convert_type(lo | (hi << 16), jnp.int32)


def _node_feat_body(x_ref, attr_ref, deg_ref, w_ref, nf_ref):
    x = x_ref[...]
    scale = attr_ref[...] * lax.rsqrt(deg_ref[...]) * (1.0 / math.sqrt(x.shape[1]))
    nf = jnp.dot(x, w_ref[...], preferred_element_type=jnp.float32) * scale
    nf_ref[...] = _pack_bf16_pair(nf)


def _edge_mlp_body(elb_ref, ea_ref, wr0_ref, wr1_ref, out_ref):
    nb = elb_ref.shape[1]
    h = jnp.dot(elb_ref[...], wr0_ref[...], preferred_element_type=jnp.float32)
    h = h * (1.0 / math.sqrt(nb))
    h = h * jax.nn.sigmoid(h)
    hdim = wr0_ref.shape[1]
    ew = jnp.dot(h, wr1_ref[...], preferred_element_type=jnp.float32)
    out_ref[...] = _pack_bf16_pair(ew * (ea_ref[...] * (1.0 / math.sqrt(hdim))))


def _final_body(x_ref, attr_ref, deg_ref, parts_ref, wm_ref, wo_ref, out_ref):
    c_s, c_x = math.sin(math.pi / 8), math.cos(math.pi / 8)
    d = x_ref.shape[1]
    attr = attr_ref[...] * (1.0 / math.sqrt(d))
    nm = jnp.dot(x_ref[...], wm_ref[...], preferred_element_type=jnp.float32) * attr
    nf2 = (parts_ref[0] + parts_ref[1]) * lax.rsqrt(deg_ref[...])
    no = jnp.dot(nf2, wo_ref[...], preferred_element_type=jnp.float32) * attr
    out_ref[...] = c_s * nm + c_x * no


def _make_sc_body(n_pad, d, ept, k_chunks):
    rps = n_pad // NS
    half = d // 2

    def body(nf_hbm, ewp_hbm, src_hbm, dst_hbm, out_hbm,
             acc, ew_v, g_v, p_v, src_v, dst_v, sem_ew, sem_g, sem_sc):
        c = lax.axis_index("c")
        s = lax.axis_index("s")
        tile = c * NS + s

        def stage_idx(grp_i):
            gb = (grp_i // GRP) % 2
            g0 = pl.multiple_of(grp_i, GRP)
            pltpu.sync_copy(src_hbm.at[tile, pl.ds(g0, GRP)], src_v.at[gb])
            pltpu.sync_copy(dst_hbm.at[tile, pl.ds(g0, GRP)], dst_v.at[gb])

        def issue_ew(t):
            b = t % 2
            e0 = tile * ept + t * CH
            pltpu.async_copy(ewp_hbm.at[pl.ds(e0, CH)], ew_v.at[b], sem_ew.at[b])

        def wait_ew(t):
            b = t % 2
            pltpu.make_async_copy(
                ewp_hbm.at[pl.ds(0, CH)], ew_v.at[b], sem_ew.at[b]).wait()

        def issue_gather(t):
            b = t % 2
            idx = src_v.at[(t // GRP) % 2, t % GRP]
            pltpu.async_copy(nf_hbm.at[idx], g_v.at[b], sem_g.at[b])

        def wait_gather(t):
            b = t % 2
            pltpu.make_async_copy(
                nf_hbm.at[src_v.at[0, 0]], g_v.at[b], sem_g.at[b]).wait()

        def issue_scatter(t):
            b = t % 2
            idx = dst_v.at[(t // GRP) % 2, t % GRP]
            pltpu.async_copy(p_v.at[b], acc.at[idx], sem_sc.at[b], add=True)

        def wait_scatter(b):
            pltpu.make_async_copy(
                p_v.at[b], acc.at[dst_v.at[0, 0]], sem_sc.at[b]).wait()

        def zrow(r, carry):
            for cc in range(d // LANES):
                p_v[0, r, pl.ds(cc * LANES, LANES)] = jnp.zeros(
                    (LANES,), jnp.float32)
            return carry
        lax.fori_loop(0, CH, zrow, 0)
        row0 = s * rps
        for kk in range(rps // CH):
            pltpu.sync_copy(p_v.at[0], acc.at[pl.ds(row0 + kk * CH, CH)])
        rem = rps % CH
        if rem:
            pltpu.sync_copy(p_v.at[0, pl.ds(0, rem)],
                            acc.at[pl.ds(row0 + (rps // CH) * CH, rem)])
        plsc.subcore_barrier()

        stage_idx(0)
        issue_ew(0)
        issue_gather(0)
        mask = jnp.int32(-65536)

        def step(j, carry):
            t = j + 1

            @pl.when(t < k_chunks)
            def _():
                @pl.when(lax.rem(t, GRP) == 0)
                def _():
                    stage_idx(t)
                issue_ew(t)
                issue_gather(t)

            wait_ew(j)
            wait_gather(j)

            @pl.when(j >= 2)
            def _():
                wait_scatter(j % 2)
            b2 = j % 2

            @plsc.parallel_loop(0, CH, 1, unroll=2)
            def mrow(r):
                for g16 in range(half // LANES):
                    lo_sl = pl.ds(g16 * LANES, LANES)
                    hi_sl = pl.ds(half + g16 * LANES, LANES)
                    ew = ew_v[b2, r, lo_sl]
                    gw = g_v[b2, r, lo_sl]
                    elo = lax.bitcast_convert_type(ew << 16, jnp.float32)
                    ehi = lax.bitcast_convert_type(ew & mask, jnp.float32)
                    glo = lax.bitcast_convert_type(gw << 16, jnp.float32)
                    ghi = lax.bitcast_convert_type(gw & mask, jnp.float32)
                    p_v[b2, r, lo_sl] = glo * elo
                    p_v[b2, r, hi_sl] = ghi * ehi

            issue_scatter(j)
            return carry
        lax.fori_loop(0, k_chunks, step, 0)
        for b in range(2):
            wait_scatter(b)

        plsc.subcore_barrier()
        pltpu.sync_copy(acc.at[pl.ds(row0, rps)], out_hbm.at[c, pl.ds(row0, rps)])

    return body


def kernel(node_input, node_attr, node_deg, edge_src, edge_dst, edge_attr,
           edge_length_embedded, numb, n, W_in, W_mask, W_r0, W_r1, W_out):
    n_nodes, d = node_input.shape
    e = edge_src.shape[0]
    nb = edge_length_embedded.shape[1]

    bn = 1000
    nf = pl.pallas_call(
        _node_feat_body,
        grid=(n_nodes // bn,),
        in_specs=[
            pl.BlockSpec((bn, d), lambda i: (i, 0)),
            pl.BlockSpec((bn, 1), lambda i: (i, 0)),
            pl.BlockSpec((bn, 1), lambda i: (i, 0)),
            pl.BlockSpec((d, d), lambda i: (0, 0)),
        ],
        out_specs=pl.BlockSpec((bn, d // 2), lambda i: (i, 0)),
        out_shape=jax.ShapeDtypeStruct((n_nodes, d // 2), jnp.int32),
    )(node_input, node_attr, node_deg, W_in)

    align = max(256, CH * GRP)
    ept = ((e + NT * align - 1) // (NT * align)) * align
    k_chunks = ept // CH
    e_pad = NT * ept
    pad = e_pad - e
    elb_p = jnp.concatenate(
        [edge_length_embedded, jnp.zeros((pad, nb), jnp.float32)])
    ea_p = jnp.concatenate([edge_attr, jnp.zeros((pad, 1), jnp.float32)])
    src_r = jnp.concatenate(
        [edge_src, jnp.zeros((pad,), jnp.int32)]).reshape(NT, k_chunks, CH)
    dst_r = jnp.concatenate(
        [edge_dst, jnp.zeros((pad,), jnp.int32)]).reshape(NT, k_chunks, CH)

    be = 8192
    hdim = W_r0.shape[1]
    ewp = pl.pallas_call(
        _edge_mlp_body,
        grid=(e_pad // be,),
        in_specs=[
            pl.BlockSpec((be, nb), lambda i: (i, 0)),
            pl.BlockSpec((be, 1), lambda i: (i, 0)),
            pl.BlockSpec((nb, hdim), lambda i: (0, 0)),
            pl.BlockSpec((hdim, d), lambda i: (0, 0)),
        ],
        out_specs=pl.BlockSpec((be, d // 2), lambda i: (i, 0)),
        out_shape=jax.ShapeDtypeStruct((e_pad, d // 2), jnp.int32),
    )(elb_p, ea_p, W_r0, W_r1)

    n_pad = ((n_nodes + NS * 8 - 1) // (NS * 8)) * (NS * 8)
    mesh = plsc.VectorSubcoreMesh(core_axis_name="c", subcore_axis_name="s")
    parts = pl.kernel(
        _make_sc_body(n_pad, d, ept, k_chunks),
        out_type=jax.ShapeDtypeStruct((NC, n_pad, d), jnp.float32),
        mesh=mesh,
        compiler_params=pltpu.CompilerParams(use_tc_tiling_on_sc=False),
        scratch_types=[
            pltpu.VMEM_SHARED((n_pad, d), jnp.float32),
            pltpu.VMEM((2, CH, d // 2), jnp.int32),
            pltpu.VMEM((2, CH, d // 2), jnp.int32),
            pltpu.VMEM((2, CH, d), jnp.float32),
            pltpu.VMEM((2, GRP, CH), jnp.int32),
            pltpu.VMEM((2, GRP, CH), jnp.int32),
            pltpu.SemaphoreType.DMA((2,)),
            pltpu.SemaphoreType.DMA((2,)),
            pltpu.SemaphoreType.DMA((2,)),
        ],
    )(nf, ewp, src_r, dst_r)

    out = pl.pallas_call(
        _final_body,
        grid=(n_nodes // bn,),
        in_specs=[
            pl.BlockSpec((bn, d), lambda i: (i, 0)),
            pl.BlockSpec((bn, 1), lambda i: (i, 0)),
            pl.BlockSpec((bn, 1), lambda i: (i, 0)),
            pl.BlockSpec((NC, bn, d), lambda i: (0, i, 0)),
            pl.BlockSpec((d, d), lambda i: (0, 0)),
            pl.BlockSpec((d, d), lambda i: (0, 0)),
        ],
        out_specs=pl.BlockSpec((bn, d), lambda i: (i, 0)),
        out_shape=jax.ShapeDtypeStruct((n_nodes, d), jnp.float32),
    )(node_input, node_attr, node_deg, parts, W_mask, W_out)
    return out
